# baseline (speedup 1.0000x reference)
import functools

import jax
import jax.numpy as jnp
from jax.experimental import pallas as pl
from jax.experimental.pallas import tpu as pltpu


def _fused_kernel(x_ref, wa_ref, o_ref):
    y = jnp.dot(wa_ref[:], x_ref[0], preferred_element_type=jnp.float32)
    o_ref[0] = y[:16] / jnp.maximum(y[16:17], 1.0)


@functools.partial(jax.jit, static_argnames=("bn",))
def _run(x, W, bn):
    b, t, vocab = x.shape
    dim = W.shape[1]
    x_t = jnp.transpose(x, (1, 2, 0))
    wa = jnp.concatenate(
        [W.T, jnp.ones((1, vocab), jnp.float32)], axis=0
    )
    grid = (t, b // bn)
    out_t = pl.pallas_call(
        _fused_kernel,
        grid=grid,
        in_specs=[
            pl.BlockSpec((1, vocab, bn), lambda i, j: (i, 0, j)),
            pl.BlockSpec((dim + 1, vocab), lambda i, j: (0, 0)),
        ],
        out_specs=pl.BlockSpec((1, dim, bn), lambda i, j: (i, 0, j)),
        out_shape=jax.ShapeDtypeStruct((t, dim, b), jnp.float32),
    )(x_t, wa)
    return jnp.transpose(out_t, (2, 0, 1))


def kernel(x_multi_hot, W):
    return _run(x_multi_hot, W, min(1024, x_multi_hot.shape[0]))


# speedup vs baseline: 1.0991x; 1.0991x over previous
import functools

import jax
import jax.numpy as jnp
from jax.experimental import pallas as pl
from jax.experimental.pallas import tpu as pltpu


def _fused_kernel(x_ref, wa_ref, o_ref):
    y = jnp.dot(wa_ref[:], x_ref[0], preferred_element_type=jnp.float32)
    o_ref[0] = y[:16] / jnp.maximum(y[16:17], 1.0)


@functools.partial(jax.jit, static_argnames=("bn",))
def _run(x, W, bn):
    b, t, vocab = x.shape
    dim = W.shape[1]
    x_t = jnp.transpose(x, (1, 2, 0))
    wa = jnp.concatenate(
        [W.T, jnp.ones((1, vocab), jnp.float32)], axis=0
    )
    grid = (t, b // bn)
    out_t = pl.pallas_call(
        _fused_kernel,
        grid=grid,
        in_specs=[
            pl.BlockSpec((1, vocab, bn), lambda i, j: (i, 0, j)),
            pl.BlockSpec((dim + 1, vocab), lambda i, j: (0, 0)),
        ],
        out_specs=pl.BlockSpec((1, dim, bn), lambda i, j: (i, 0, j)),
        out_shape=jax.ShapeDtypeStruct((t, dim, b), jnp.float32),
    )(x_t, wa)
    return jnp.transpose(out_t, (2, 0, 1))


def kernel(x_multi_hot, W):
    return _run(x_multi_hot, W, min(4096, x_multi_hot.shape[0]))


# speedup vs baseline: 1.1338x; 1.0316x over previous
import functools

import jax
import jax.numpy as jnp
from jax.experimental import pallas as pl
from jax.experimental.pallas import tpu as pltpu


def _fused_kernel(x_ref, wa_ref, o_ref):
    y = jnp.dot(wa_ref[:], x_ref[0], preferred_element_type=jnp.float32)
    o_ref[0] = y[:16] / jnp.maximum(y[16:17], 1.0)


@functools.partial(jax.jit, static_argnames=("bn",))
def _run(x, W, bn):
    b, t, vocab = x.shape
    dim = W.shape[1]
    x_t = jnp.transpose(x, (1, 2, 0))
    wa = jnp.concatenate(
        [W.T, jnp.ones((1, vocab), jnp.float32)], axis=0
    )
    grid = (t, b // bn)
    out_t = pl.pallas_call(
        _fused_kernel,
        grid=grid,
        in_specs=[
            pl.BlockSpec((1, vocab, bn), lambda i, j: (i, 0, j)),
            pl.BlockSpec((dim + 1, vocab), lambda i, j: (0, 0)),
        ],
        out_specs=pl.BlockSpec((1, dim, bn), lambda i, j: (i, 0, j)),
        out_shape=jax.ShapeDtypeStruct((t, dim, b), jnp.float32),
        compiler_params=pltpu.CompilerParams(
            dimension_semantics=("parallel", "parallel"),
        ),
    )(x_t, wa)
    return jnp.transpose(out_t, (2, 0, 1))


def kernel(x_multi_hot, W):
    return _run(x_multi_hot, W, min(2048, x_multi_hot.shape[0]))


# speedup vs baseline: 1.1515x; 1.0155x over previous
import functools

import jax
import jax.numpy as jnp
from jax.experimental import pallas as pl
from jax.experimental.pallas import tpu as pltpu


def _fused_kernel(x_ref, wt_ref, o_ref):
    x = x_ref[0]
    y = jnp.dot(wt_ref[:], x, preferred_element_type=jnp.float32)
    s = jnp.sum(x, axis=0, keepdims=True)
    o_ref[0] = y / jnp.maximum(s, 1.0)


@functools.partial(jax.jit, static_argnames=("bn",))
def _run(x, W, bn):
    b, t, vocab = x.shape
    dim = W.shape[1]
    x_t = jnp.transpose(x, (1, 2, 0))
    wt = W.T
    grid = (t, b // bn)
    out_t = pl.pallas_call(
        _fused_kernel,
        grid=grid,
        in_specs=[
            pl.BlockSpec((1, vocab, bn), lambda i, j: (i, 0, j)),
            pl.BlockSpec((dim, vocab), lambda i, j: (0, 0)),
        ],
        out_specs=pl.BlockSpec((1, dim, bn), lambda i, j: (i, 0, j)),
        out_shape=jax.ShapeDtypeStruct((t, dim, b), jnp.float32),
        compiler_params=pltpu.CompilerParams(
            dimension_semantics=("parallel", "parallel"),
        ),
    )(x_t, wt)
    return jnp.transpose(out_t, (2, 0, 1))


def kernel(x_multi_hot, W):
    return _run(x_multi_hot, W, min(2048, x_multi_hot.shape[0]))
